# baseline (speedup 1.0000x reference)
import functools
import math

import jax
import jax.numpy as jnp
from jax import lax
from jax.experimental import pallas as pl
from jax.experimental.pallas import tpu as pltpu
from jax.experimental.pallas import tpu_sc as plsc

VOCAB = 1000000
EMB = 64
SCALE = math.sqrt(EMB)

_NUM_CORES = 2
_NUM_SUBCORES = 16
_NW = _NUM_CORES * _NUM_SUBCORES

_BATCH = 4096
_SEQ = 200
_HSEQ = _SEQ // 2
_IDXROW = 2 * _HSEQ + 8
_GLEN = _HSEQ + 4
_ODD_OFF = _HSEQ + 4
_B_PER_W = _BATCH // _NW
_NBUF = 3
_HEAD = 5


def _sc_embed(tokens_arr, table):
    mesh = plsc.VectorSubcoreMesh(
        core_axis_name="c", subcore_axis_name="s")

    @functools.partial(
        pl.kernel,
        out_type=jax.ShapeDtypeStruct((_BATCH, _HSEQ, 2 * EMB), jnp.float32),
        mesh=mesh,
        scratch_types=[
            pltpu.VMEM((_B_PER_W, _IDXROW), jnp.int32),
            [pltpu.VMEM((_GLEN, EMB), jnp.float32)] * _NBUF,
            [pltpu.VMEM((_GLEN, EMB), jnp.float32)] * _NBUF,
            [pltpu.VMEM((_HSEQ, 2 * EMB), jnp.float32)] * _NBUF,
            [pltpu.SemaphoreType.DMA] * _NBUF,
            [pltpu.SemaphoreType.DMA] * _NBUF,
            [pltpu.SemaphoreType.DMA] * _NBUF,
        ],
        compiler_params=pltpu.CompilerParams(use_tc_tiling_on_sc=False),
    )
    def body(tok_hbm, table_hbm, out_hbm, idx_all,
             bufe, bufo, pair, esem, osem, wsem):
        wid = lax.axis_index("s") * _NUM_CORES + lax.axis_index("c")
        base = wid * _B_PER_W

        pltpu.sync_copy(tok_hbm.at[pl.ds(base, _B_PER_W)], idx_all)

        def gathers(g, b):
            return (
                pltpu.make_async_copy(
                    table_hbm.at[idx_all.at[g, pl.ds(0, _GLEN)]],
                    bufe[b], esem[b]),
                pltpu.make_async_copy(
                    table_hbm.at[idx_all.at[g, pl.ds(_ODD_OFF, _GLEN)]],
                    bufo[b], osem[b]),
            )

        def write(g, b):
            return pltpu.make_async_copy(
                pair[b], out_hbm.at[base + g], wsem[b])

        def scale_interleave(b):
            e, o, p = bufe[b], bufo[b], pair[b]

            @plsc.parallel_loop(0, _HSEQ, unroll=4)
            def _(i):
                for j in range(EMB // 16):
                    sl = pl.ds(j * 16, 16)
                    so = pl.ds(EMB + j * 16, 16)
                    p[i, sl] = e[i, sl] * SCALE
                    p[i, so] = o[i, sl] * SCALE

        def step(g, p, wait_write, prefetch):
            ge, go = gathers(g, p)
            ge.wait()
            go.wait()
            scale_interleave(p)
            write(g, p).start()
            if prefetch:
                f = g + 1
                q = (p + 1) % _NBUF
                if wait_write:
                    write(f - _NBUF, q).wait()
                fe, fo = gathers(f, q)
                fe.start()
                fo.start()

        ge0, go0 = gathers(0, 0)
        ge0.start()
        go0.start()
        for g in range(_HEAD):
            step(g, g % _NBUF, wait_write=(g >= _NBUF - 1), prefetch=True)
        nblocks = (_B_PER_W - 3 - _HEAD) // _NBUF

        def block(G, carry):
            for b in range(_NBUF):
                g = _HEAD + G * _NBUF + b
                step(g, (_HEAD + b) % _NBUF, wait_write=True, prefetch=True)
            return carry

        lax.fori_loop(0, nblocks, block, 0)
        for g in range(_B_PER_W - 3, _B_PER_W - 1):
            step(g, g % _NBUF, wait_write=True, prefetch=True)
        step(_B_PER_W - 1, (_B_PER_W - 1) % _NBUF,
             wait_write=False, prefetch=False)
        for g in range(_B_PER_W - _NBUF, _B_PER_W):
            write(g, g % _NBUF).wait()

    return body(tokens_arr, table)


def kernel(tokens, table):
    tok = tokens.astype(jnp.int32)
    pad = jnp.zeros((_BATCH, 4), jnp.int32)
    arranged = jnp.concatenate(
        [tok[:, 0::2], pad, tok[:, 1::2], pad], axis=1)
    out = _sc_embed(arranged, table)
    return out.reshape(_BATCH, _SEQ, EMB)


# speedup vs baseline: 1.5573x; 1.5573x over previous
import functools
import math

import jax
import jax.numpy as jnp
from jax import lax
from jax.experimental import pallas as pl
from jax.experimental.pallas import tpu as pltpu
from jax.experimental.pallas import tpu_sc as plsc

VOCAB = 1000000
EMB = 64
SCALE = math.sqrt(EMB)

_NUM_CORES = 2
_NUM_SUBCORES = 16
_NW = _NUM_CORES * _NUM_SUBCORES

_B = 4096 * 200
_CHUNK = 128
_NPAIR = _CHUNK // 2
_NCHUNK = _B // (_NW * _CHUNK)
_NGB = 3
_NPB = 2
_LOOK = 2


def _sc_embed(tokens_w, table_pad):
    mesh = plsc.VectorSubcoreMesh(
        core_axis_name="c", subcore_axis_name="s")

    @functools.partial(
        pl.kernel,
        out_type=jax.ShapeDtypeStruct((_B // 2, 2 * EMB), jnp.float32),
        mesh=mesh,
        scratch_types=[
            pltpu.VMEM((_NCHUNK, _CHUNK), jnp.int32),
            [pltpu.VMEM((_CHUNK, 2 * EMB), jnp.float32)] * _NGB,
            [pltpu.VMEM((_NPAIR, 2 * EMB), jnp.float32)] * _NPB,
            [pltpu.SemaphoreType.DMA] * _NGB,
            [pltpu.SemaphoreType.DMA] * _NPB,
        ],
        compiler_params=pltpu.CompilerParams(use_tc_tiling_on_sc=True),
    )
    def body(tok_hbm, table_hbm, out_hbm, idx_all, rows, pair, gsem, wsem):
        wid = lax.axis_index("s") * _NUM_CORES + lax.axis_index("c")
        pbase = wid * (_NCHUNK * _NPAIR)

        pltpu.sync_copy(tok_hbm.at[wid], idx_all)

        def gather(g, b):
            return pltpu.make_async_copy(
                table_hbm.at[idx_all.at[g]], rows[b], gsem[b])

        def write(g, b):
            return pltpu.make_async_copy(
                pair[b], out_hbm.at[pl.ds(pbase + g * _NPAIR, _NPAIR)],
                wsem[b])

        def interleave(gb, pb):
            r, p = rows[gb], pair[pb]

            @plsc.parallel_loop(0, _NPAIR, unroll=4)
            def _(i):
                i2 = 2 * i
                for j in range(EMB // 16):
                    sl = pl.ds(j * 16, 16)
                    so = pl.ds(EMB + j * 16, 16)
                    p[i, sl] = r[i2, sl] * SCALE
                    p[i, so] = r[i2 + 1, sl] * SCALE

        def step(g, gb, pb, pfb, wait_write, prefetch):
            gather(g, gb).wait()
            interleave(gb, pb)
            if wait_write:
                write(g - _NPB, pb).wait()
            write(g, pb).start()
            if prefetch:
                gather(g + _LOOK, pfb).start()

        for j in range(_LOOK):
            gather(j, j).start()
        for g in range(_LOOK):
            step(g, g % _NGB, g % _NPB, (g + _LOOK) % _NGB,
                 wait_write=False, prefetch=True)
        nblocks = (_NCHUNK - _LOOK - 6) // 6

        def block(G, carry):
            for b in range(6):
                g = _LOOK + G * 6 + b
                step(g, (_LOOK + b) % _NGB, b % _NPB,
                     (2 * _LOOK + b) % _NGB,
                     wait_write=True, prefetch=True)
            return carry

        lax.fori_loop(0, nblocks, block, 0)
        for g in range(_NCHUNK - 6, _NCHUNK - _LOOK):
            step(g, g % _NGB, g % _NPB, (g + _LOOK) % _NGB,
                 wait_write=True, prefetch=True)
        for g in range(_NCHUNK - _LOOK, _NCHUNK):
            step(g, g % _NGB, g % _NPB, 0,
                 wait_write=True, prefetch=False)
        for g in range(_NCHUNK - _NPB, _NCHUNK):
            write(g, g % _NPB).wait()

    return body(tokens_w, table_pad)


def kernel(tokens, table):
    tok = tokens.astype(jnp.int32).reshape(_NW, _NCHUNK, _CHUNK)
    table_pad = jnp.pad(table, ((0, 0), (0, 2 * EMB - table.shape[1])))
    out = _sc_embed(tok, table_pad)
    return out.reshape(tokens.shape + (EMB,))


# speedup vs baseline: 1.5601x; 1.0018x over previous
import functools
import math

import jax
import jax.numpy as jnp
from jax import lax
from jax.experimental import pallas as pl
from jax.experimental.pallas import tpu as pltpu
from jax.experimental.pallas import tpu_sc as plsc

VOCAB = 1000000
EMB = 64
SCALE = math.sqrt(EMB)

_NUM_CORES = 2
_NUM_SUBCORES = 16
_NW = _NUM_CORES * _NUM_SUBCORES

_B = 4096 * 200
_CHUNK = 128
_NPAIR = _CHUNK // 2
_NCHUNK = _B // (_NW * _CHUNK)
_NGB = 3
_NPB = 2
_LOOK = 2


def _sc_embed(tokens_w, table_pad):
    mesh = plsc.VectorSubcoreMesh(
        core_axis_name="c", subcore_axis_name="s")

    @functools.partial(
        pl.kernel,
        out_type=jax.ShapeDtypeStruct((_B // 2, 2 * EMB), jnp.float32),
        mesh=mesh,
        scratch_types=[
            pltpu.VMEM((_NCHUNK, _CHUNK), jnp.int32),
            [pltpu.VMEM((_CHUNK, 2 * EMB), jnp.float32)] * _NGB,
            [pltpu.VMEM((_NPAIR, 2 * EMB), jnp.float32)] * _NPB,
            [pltpu.SemaphoreType.DMA] * _NGB,
            [pltpu.SemaphoreType.DMA] * _NPB,
        ],
        compiler_params=pltpu.CompilerParams(
            use_tc_tiling_on_sc=True, needs_layout_passes=True),
    )
    def body(tok_hbm, table_hbm, out_hbm, idx_all, rows, pair, gsem, wsem):
        wid = lax.axis_index("s") * _NUM_CORES + lax.axis_index("c")
        pbase = wid * (_NCHUNK * _NPAIR)

        pltpu.sync_copy(tok_hbm.at[wid], idx_all)

        def gather(g, b):
            return pltpu.make_async_copy(
                table_hbm.at[idx_all.at[g]], rows[b], gsem[b])

        def write(g, b):
            return pltpu.make_async_copy(
                pair[b], out_hbm.at[pl.ds(pbase + g * _NPAIR, _NPAIR)],
                wsem[b])

        def interleave(gb, pb):
            r, p = rows[gb], pair[pb]

            @plsc.parallel_loop(0, _NPAIR, unroll=4)
            def _(i):
                i2 = 2 * i
                for j in range(EMB // 16):
                    sl = pl.ds(j * 16, 16)
                    so = pl.ds(EMB + j * 16, 16)
                    p[i, sl] = r[i2, sl] * SCALE
                    p[i, so] = r[i2 + 1, sl] * SCALE

        def step(g, gb, pb, pfb, wait_write, prefetch):
            gather(g, gb).wait()
            interleave(gb, pb)
            if wait_write:
                write(g - _NPB, pb).wait()
            write(g, pb).start()
            if prefetch:
                gather(g + _LOOK, pfb).start()

        for j in range(_LOOK):
            gather(j, j).start()
        for g in range(_LOOK):
            step(g, g % _NGB, g % _NPB, (g + _LOOK) % _NGB,
                 wait_write=False, prefetch=True)
        nblocks = (_NCHUNK - _LOOK - 6) // 6

        def block(G, carry):
            for b in range(6):
                g = _LOOK + G * 6 + b
                step(g, (_LOOK + b) % _NGB, b % _NPB,
                     (2 * _LOOK + b) % _NGB,
                     wait_write=True, prefetch=True)
            return carry

        lax.fori_loop(0, nblocks, block, 0)
        for g in range(_NCHUNK - 6, _NCHUNK - _LOOK):
            step(g, g % _NGB, g % _NPB, (g + _LOOK) % _NGB,
                 wait_write=True, prefetch=True)
        for g in range(_NCHUNK - _LOOK, _NCHUNK):
            step(g, g % _NGB, g % _NPB, 0,
                 wait_write=True, prefetch=False)
        for g in range(_NCHUNK - _NPB, _NCHUNK):
            write(g, g % _NPB).wait()

    return body(tokens_w, table_pad)


def kernel(tokens, table):
    tok = tokens.astype(jnp.int32).reshape(_NW, _NCHUNK, _CHUNK)
    table_pad = jnp.pad(table, ((0, 0), (0, 2 * EMB - table.shape[1])))
    out = _sc_embed(tok, table_pad)
    return out.reshape(tokens.shape + (EMB,))
